# Initial kernel scaffold; baseline (speedup 1.0000x reference)
import jax
import jax.numpy as jnp
from jax.experimental import pallas as pl


def kernel(F_0, edge_index, meshfield, enc_Wl1, enc_Wr1, enc_b1, enc_Wl2, enc_Wr2, enc_b2, dec_Wl1, dec_Wr1, dec_b1, dec_Wl2, dec_Wr2, dec_b2, lin_W, lin_b):
    raise NotImplementedError("write your pallas kernel here")


# speedup vs baseline: 5.0651x; 5.0651x over previous
import functools
import jax
import jax.numpy as jnp
from jax import lax
from jax.experimental import pallas as pl
from jax.experimental.pallas import tpu as pltpu
from jax.experimental.pallas import tpu_sc as plsc

N = 10000
E = 320000
H = 128

NC = 2
NS = 16
NW = NC * NS
EPW = E // NW
C = 128
K = (EPW + C - 1) // C
EPW_PAD = K * C
ZR = 632
ACC_ROWS = NS * ZR


def _make_spmm(D):
    mesh = plsc.VectorSubcoreMesh(core_axis_name="c", subcore_axis_name="s")

    @functools.partial(
        pl.kernel,
        mesh=mesh,
        out_type=jax.ShapeDtypeStruct((NC, N, D), jnp.float32),
        scratch_types=[
            pltpu.VMEM((K, C), jnp.int32),
            pltpu.VMEM((K, C), jnp.int32),
            pltpu.VMEM((C, D), jnp.float32),
            pltpu.VMEM_SHARED((ACC_ROWS, D), jnp.float32),
            pltpu.SemaphoreType.DMA,
        ],
    )
    def spmm(z_hbm, src_hbm, dst_hbm, zeros_hbm, out_hbm,
             src_v, dst_v, rows_v, acc_sh, sem):
        cid = lax.axis_index("c")
        sid = lax.axis_index("s")
        wid = cid * NS + sid
        pltpu.sync_copy(zeros_hbm, acc_sh.at[pl.ds(sid * ZR, ZR), :])
        pltpu.sync_copy(src_hbm.at[wid], src_v)
        pltpu.sync_copy(dst_hbm.at[wid], dst_v)
        plsc.subcore_barrier()

        def body(j, carry):
            pltpu.async_copy(z_hbm.at[src_v.at[j]], rows_v, sem).wait()
            pltpu.sync_copy(rows_v, acc_sh.at[dst_v.at[j]], add=True)
            return carry

        lax.fori_loop(0, K, body, 0)
        plsc.subcore_barrier()

        @pl.when(sid < NS - 1)
        def _():
            pltpu.sync_copy(acc_sh.at[pl.ds(sid * ZR, ZR), :],
                            out_hbm.at[cid, pl.ds(sid * ZR, ZR), :])

        @pl.when(sid == NS - 1)
        def _():
            tail = N - (NS - 1) * ZR
            pltpu.sync_copy(acc_sh.at[pl.ds((NS - 1) * ZR, tail), :],
                            out_hbm.at[cid, pl.ds((NS - 1) * ZR, tail), :])

    return spmm


_spmm128 = _make_spmm(H)

BN = 1000
GRID = N // BN


def _row_spec(d):
    return pl.BlockSpec((BN, d), lambda i: (i, 0))


def _full_spec(r, c):
    return pl.BlockSpec((r, c), lambda i: (0, 0))


def _tc1_body(s0, s1, x7, wl, wr, b, wnext, x1_o, z2_o, inv_o):
    s = s0[...] + s1[...]
    inv = 1.0 / jnp.maximum(s[:, 7:8], 1.0)
    aggr = s * inv
    x1 = jnp.maximum(
        jnp.dot(aggr, wl[...], preferred_element_type=jnp.float32) + b[...]
        + jnp.dot(x7[...], wr[...], preferred_element_type=jnp.float32), 0.0)
    x1_o[...] = x1
    z2_o[...] = jnp.dot(x1, wnext[...], preferred_element_type=jnp.float32)
    inv_o[...] = inv


def _tc1(s0, s1, x7, wl, wr, b, wnext):
    return pl.pallas_call(
        _tc1_body,
        grid=(GRID,),
        in_specs=[_row_spec(H), _row_spec(H), _row_spec(H),
                  _full_spec(H, H), _full_spec(H, H), _full_spec(1, H),
                  _full_spec(H, H)],
        out_specs=[_row_spec(H), _row_spec(H), _row_spec(1)],
        out_shape=[jax.ShapeDtypeStruct((N, H), jnp.float32),
                   jax.ShapeDtypeStruct((N, H), jnp.float32),
                   jax.ShapeDtypeStruct((N, 1), jnp.float32)],
    )(s0, s1, x7, wl, wr, b, wnext)


def _tcmid_body(s0, s1, inv, x, wr, b, wnext, xn_o, zn_o):
    xn = jnp.maximum(
        (s0[...] + s1[...]) * inv[...] + b[...]
        + jnp.dot(x[...], wr[...], preferred_element_type=jnp.float32), 0.0)
    xn_o[...] = xn
    zn_o[...] = jnp.dot(xn, wnext[...], preferred_element_type=jnp.float32)


def _tcmid(s0, s1, inv, x, wr, b, wnext):
    return pl.pallas_call(
        _tcmid_body,
        grid=(GRID,),
        in_specs=[_row_spec(H), _row_spec(H), _row_spec(1), _row_spec(H),
                  _full_spec(H, H), _full_spec(1, H), _full_spec(H, H)],
        out_specs=[_row_spec(H), _row_spec(H)],
        out_shape=[jax.ShapeDtypeStruct((N, H), jnp.float32),
                   jax.ShapeDtypeStruct((N, H), jnp.float32)],
    )(s0, s1, inv, x, wr, b, wnext)


def _tc4_body(s0, s1, inv, x, wr, b, lw, lb, xprev, out_o):
    x4 = jnp.maximum(
        (s0[...] + s1[...]) * inv[...] + b[...]
        + jnp.dot(x[...], wr[...], preferred_element_type=jnp.float32), 0.0)
    out_o[...] = xprev[...] + jnp.dot(
        x4, lw[...], preferred_element_type=jnp.float32) + lb[...]


def _tc4(s0, s1, inv, x, wr, b, lw, lb, xprev):
    return pl.pallas_call(
        _tc4_body,
        grid=(GRID,),
        in_specs=[_row_spec(H), _row_spec(H), _row_spec(1), _row_spec(H),
                  _full_spec(H, H), _full_spec(1, H), _full_spec(H, 4),
                  _full_spec(1, 4), _row_spec(4)],
        out_specs=_row_spec(4),
        out_shape=jax.ShapeDtypeStruct((N, 4), jnp.float32),
    )(s0, s1, inv, x, wr, b, lw, lb, xprev)


@jax.jit
def kernel(F_0, edge_index, meshfield,
           enc_Wl1, enc_Wr1, enc_b1, enc_Wl2, enc_Wr2, enc_b2,
           dec_Wl1, dec_Wr1, dec_b1, dec_Wl2, dec_Wr2, dec_b2,
           lin_W, lin_b):
    x_prev = F_0[:, -1]
    x7 = jnp.concatenate([F_0.reshape(N, -1), meshfield], axis=1)
    z1 = jnp.concatenate(
        [x7, jnp.ones((N, 1), jnp.float32),
         jnp.zeros((N, H - 8), jnp.float32)], axis=1)
    x7p = jnp.concatenate([x7, jnp.zeros((N, H - 7), jnp.float32)], axis=1)

    def padH(w):
        return jnp.concatenate([w, jnp.zeros((H - 7, H), jnp.float32)], axis=0)

    wl1 = padH(enc_Wl1)
    wr1 = padH(enc_Wr1)

    src = edge_index[0].reshape(NW, EPW)
    dst = edge_index[1].reshape(NW, EPW)
    padn = EPW_PAD - EPW
    src3 = jnp.concatenate(
        [src, jnp.zeros((NW, padn), jnp.int32)], axis=1).reshape(NW, K, C)
    dst3 = jnp.concatenate(
        [dst, jnp.full((NW, padn), N, jnp.int32)], axis=1).reshape(NW, K, C)

    zeros128 = jnp.zeros((ZR, H), jnp.float32)

    s1p = _spmm128(z1, src3, dst3, zeros128)
    x1, z2, inv = _tc1(s1p[0], s1p[1], x7p, wl1, wr1,
                       enc_b1.reshape(1, H), enc_Wl2)
    s2p = _spmm128(z2, src3, dst3, zeros128)
    x2, z3 = _tcmid(s2p[0], s2p[1], inv, x1, enc_Wr2,
                    enc_b2.reshape(1, H), dec_Wl1)
    s3p = _spmm128(z3, src3, dst3, zeros128)
    x3, z4 = _tcmid(s3p[0], s3p[1], inv, x2, dec_Wr1,
                    dec_b1.reshape(1, H), dec_Wl2)
    s4p = _spmm128(z4, src3, dst3, zeros128)
    out = _tc4(s4p[0], s4p[1], inv, x3, dec_Wr2,
               dec_b2.reshape(1, H), lin_W, lin_b.reshape(1, 4), x_prev)
    return out[:, None, :]
